# baseline (speedup 1.0000x reference)
import functools

import jax
import jax.numpy as jnp
from jax import lax
from jax.experimental import pallas as pl
from jax.experimental.pallas import tpu as pltpu
from jax.experimental.pallas import tpu_sc as plsc

F = 26
V = 100000
E = 16
ND = 11
B = 16384
H1, H2, OUT = 128, 64, 10

NC, NS = 2, 16
NW = NC * NS
IPW = B // NW
IB = 128
NBLK = IPW // IB
HFLD = 13
CH = HFLD * IB


def _sc_body(table, idxs, bi_out, idx_v, rows_v, sum_v, ssq_v, sem):
    wid = lax.axis_index("s") * NC + lax.axis_index("c")
    for blk in range(NBLK):
        blkg = wid * NBLK + blk
        item0 = blkg * IB
        for h in range(2):
            pltpu.sync_copy(
                idxs.at[pl.ds((blkg * F + h * HFLD) * IB, CH)], idx_v)
            descs = []
            for j in range(HFLD):
                descs.append(pltpu.async_copy(
                    table.at[idx_v.at[pl.ds(j * IB, IB)]],
                    rows_v.at[pl.ds(j * IB, IB), :], sem))
            for d in descs:
                d.wait()

            if h == 0:
                def body0(b, c):
                    v = rows_v[b]
                    s = v
                    q = v * v
                    for f in range(1, HFLD):
                        v = rows_v[f * IB + b]
                        s = s + v
                        q = q + v * v
                    sum_v[b] = s
                    ssq_v[b] = q
                    return c
                lax.fori_loop(0, IB, body0, 0)
            else:
                def body1(b, c):
                    v = rows_v[b]
                    s = v
                    q = v * v
                    for f in range(1, HFLD):
                        v = rows_v[f * IB + b]
                        s = s + v
                        q = q + v * v
                    st = sum_v[b] + s
                    qt = ssq_v[b] + q
                    sum_v[b] = 0.5 * (st * st - qt)
                    return c
                lax.fori_loop(0, IB, body1, 0)
        pltpu.sync_copy(sum_v, bi_out.at[pl.ds(item0, IB), :])


_sc_pool = functools.partial(
    pl.kernel,
    out_type=jax.ShapeDtypeStruct((B, E), jnp.float32),
    mesh=plsc.VectorSubcoreMesh(core_axis_name="c", subcore_axis_name="s"),
    scratch_types=[
        pltpu.VMEM((CH,), jnp.int32),
        pltpu.VMEM((CH, E), jnp.float32),
        pltpu.VMEM((IB, E), jnp.float32),
        pltpu.VMEM((IB, E), jnp.float32),
        pltpu.SemaphoreType.DMA,
    ],
    compiler_params=pltpu.CompilerParams(use_tc_tiling_on_sc=False),
)(_sc_body)


BM = 2048


def _mlp_body(dense_ref, bi_ref, w1a_ref, w1b_ref, b1_ref, w2_ref, b2_ref,
              w3_ref, b3_ref, out_ref):
    h = jnp.dot(dense_ref[...], w1a_ref[...], preferred_element_type=jnp.float32)
    h += jnp.dot(bi_ref[...], w1b_ref[...], preferred_element_type=jnp.float32)
    h = jnp.maximum(h + b1_ref[...], 0.0)
    h = jnp.dot(h, w2_ref[...], preferred_element_type=jnp.float32)
    h = jnp.maximum(h + b2_ref[...], 0.0)
    out_ref[...] = (
        jnp.dot(h, w3_ref[...], preferred_element_type=jnp.float32)
        + b3_ref[...])


def _mlp(dense, bi, W1a, W1b, b1, W2, b2, W3, b3):
    grid = (B // BM,)
    full = lambda shape: pl.BlockSpec(shape, lambda i: (0, 0))
    return pl.pallas_call(
        _mlp_body,
        grid=grid,
        in_specs=[
            pl.BlockSpec((BM, ND), lambda i: (i, 0)),
            pl.BlockSpec((BM, E), lambda i: (i, 0)),
            full((ND, H1)),
            full((E, H1)),
            full((1, H1)),
            full((H1, H2)),
            full((1, H2)),
            full((H2, OUT)),
            full((1, OUT)),
        ],
        out_specs=pl.BlockSpec((BM, OUT), lambda i: (i, 0)),
        out_shape=jax.ShapeDtypeStruct((B, OUT), jnp.float32),
    )(dense, bi, W1a, W1b, b1, W2, b2, W3, b3)


def kernel(target_x, tables, W1, b1, W2, b2, W3, b3):
    dense = target_x[:, :ND]
    sparse = target_x[:, ND:].astype(jnp.int32)
    flat_idx = sparse + (jnp.arange(F, dtype=jnp.int32) * V)[None, :]
    idx_blocks = flat_idx.reshape(B // IB, IB, F).transpose(0, 2, 1).reshape(-1)
    table_flat = tables.reshape(F * V, E)

    bi = _sc_pool(table_flat, idx_blocks)

    return _mlp(dense, bi, W1[:ND], W1[ND:], b1[None, :], W2, b2[None, :],
                W3, b3[None, :])
